# baseline (speedup 1.0000x reference)
import jax
import jax.numpy as jnp
from jax.experimental import pallas as pl

N = 10000
P0 = 0.5


def _copy_body(x_ref, o_ref):
    o_ref[...] = x_ref[...]


def _att(h, row, col):
    norms = jnp.sqrt(jnp.sum(h * h, axis=1))
    safe = jnp.where(norms == 0, 1.0, norms)
    sim = jnp.sum(h[row] * h[col], axis=1) / (safe[row] * safe[col])
    sim = jnp.where(sim < P0, 0.0, sim)
    row_sum = jax.ops.segment_sum(sim, row, num_segments=N)
    denom = jnp.where(row_sum == 0, 1.0, row_sum)
    w = sim / denom[row]
    deg = jax.ops.segment_sum((w > 0).astype(jnp.float32), row, num_segments=N)
    lam = 1.0 / (deg + 1.0)
    ew = jnp.where(w > 0, jnp.exp(w), 0.0)
    ar = jnp.arange(N, dtype=row.dtype)
    return (jnp.concatenate([row, ar]), jnp.concatenate([col, ar]),
            jnp.concatenate([ew, jnp.exp(lam)]))


def _conv(h, r, c, ew, W, b):
    deg = jax.ops.segment_sum(ew, c, num_segments=N)
    dis = jnp.where(deg > 0, 1.0 / jnp.sqrt(jnp.where(deg > 0, deg, 1.0)), 0.0)
    norm = dis[r] * ew * dis[c]
    h2 = h @ W
    out = jax.ops.segment_sum(norm[:, None] * h2[r], c, num_segments=N)
    return out + b


def kernel(x, adj_indices, W1, b1, W2, b2, W3, b3):
    row, col = adj_indices[0], adj_indices[1]
    h = x
    for (W, b) in ((W1, b1), (W2, b2)):
        r, c, w = _att(h, row, col)
        h = jax.nn.relu(_conv(h, r, c, w, W, b))
    r, c, w = _att(h, row, col)
    h = _conv(h, r, c, w, W3, b3)
    out = jax.nn.log_softmax(h, axis=1)
    return pl.pallas_call(
        _copy_body,
        out_shape=jax.ShapeDtypeStruct(out.shape, out.dtype),
    )(out)


# speedup vs baseline: 5.6598x; 5.6598x over previous
import functools

import jax
import jax.numpy as jnp
from jax import lax
from jax.experimental import pallas as pl
from jax.experimental.pallas import tpu as pltpu
from jax.experimental.pallas import tpu_sc as plsc

N = 10000
E = 320000
P0 = 0.5
NC = 2
NS = 16
NW = NC * NS
EPW = E // NW
C = 80
G = C // 16
NCH = EPW // C

PF = 128

F32 = jnp.float32
I32 = jnp.int32


def _prep_body(h_ref, w_ref, xn_ref, hw_ref):
    h = h_ref[...]
    n2 = jnp.sum(h * h, axis=1, keepdims=True)
    safe = jnp.where(n2 == 0.0, 1.0, jnp.sqrt(n2))
    xn = h / safe
    F = h.shape[1]
    if F < PF:
        xn = jnp.concatenate([xn, jnp.zeros((h.shape[0], PF - F), F32)], axis=1)
    xn_ref[...] = xn
    hw_ref[...] = jnp.dot(h, w_ref[...], preferred_element_type=F32)


def _prep(h, W):
    F, Fo = h.shape[1], W.shape[1]
    BN = 2000
    return pl.pallas_call(
        _prep_body,
        grid=(N // BN,),
        in_specs=[pl.BlockSpec((BN, F), lambda i: (i, 0)),
                  pl.BlockSpec((F, Fo), lambda i: (0, 0))],
        out_specs=[pl.BlockSpec((BN, PF), lambda i: (i, 0)),
                   pl.BlockSpec((BN, Fo), lambda i: (i, 0))],
        out_shape=[jax.ShapeDtypeStruct((N, PF), F32),
                   jax.ShapeDtypeStruct((N, Fo), F32)],
    )(h, W)


def _mid_body(rs_ref, dg_ref, den_ref, sl_ref):
    rs = jnp.sum(rs_ref[...], axis=0, keepdims=True)
    dg = jnp.sum(dg_ref[...], axis=0, keepdims=True)
    den_ref[...] = jnp.where(rs == 0.0, 1.0, rs)
    sl_ref[...] = jnp.exp(1.0 / (dg + 1.0))


def _mid(rs2, dg2):
    return pl.pallas_call(
        _mid_body,
        out_shape=[jax.ShapeDtypeStruct((1, N), F32),
                   jax.ShapeDtypeStruct((1, N), F32)],
    )(rs2, dg2)


def _post1_body(dc_ref, sl_ref, dis_ref):
    degc = jnp.sum(dc_ref[...], axis=0, keepdims=True) + sl_ref[...]
    pos = degc > 0.0
    dis_ref[...] = jnp.where(pos, lax.rsqrt(jnp.where(pos, degc, 1.0)), 0.0)


def _post1(dc2, slw):
    return pl.pallas_call(
        _post1_body,
        out_shape=jax.ShapeDtypeStruct((1, N), F32),
    )(dc2, slw)


def _post2_body(hw_ref, dis_ref, sl_ref, hws_ref, so_ref):
    hw = hw_ref[...]
    dis = dis_ref[...]
    hws = hw * dis
    Fo = hw.shape[1]
    if Fo < PF:
        hws = jnp.concatenate(
            [hws, jnp.zeros((hw.shape[0], PF - Fo), F32)], axis=1)
    hws_ref[...] = hws
    so_ref[...] = hw * (dis * dis * sl_ref[...])


def _post2(hw, dis_col, sl_col):
    Fo = hw.shape[1]
    BN = 2000
    return pl.pallas_call(
        _post2_body,
        grid=(N // BN,),
        in_specs=[pl.BlockSpec((BN, Fo), lambda i: (i, 0)),
                  pl.BlockSpec((BN, 1), lambda i: (i, 0)),
                  pl.BlockSpec((BN, 1), lambda i: (i, 0))],
        out_specs=[pl.BlockSpec((BN, PF), lambda i: (i, 0)),
                   pl.BlockSpec((BN, Fo), lambda i: (i, 0))],
        out_shape=[jax.ShapeDtypeStruct((N, PF), F32),
                   jax.ShapeDtypeStruct((N, Fo), F32)],
    )(hw, dis_col, sl_col)


def _fin_body(m_ref, so_ref, b_ref, o_ref, *, act):
    Fo = so_ref.shape[1]
    m = m_ref[...]
    s = m[0, :, :Fo] + m[1, :, :Fo] + so_ref[...] + b_ref[...]
    if act == "relu":
        o_ref[...] = jnp.maximum(s, 0.0)
    elif act == "lsm":
        mx = jnp.max(s, axis=1, keepdims=True)
        z = s - mx
        o_ref[...] = z - jnp.log(jnp.sum(jnp.exp(z), axis=1, keepdims=True))
    else:
        o_ref[...] = s


def _fin(m2, so, b_row, act):
    Fo = so.shape[1]
    BN = 2000
    return pl.pallas_call(
        functools.partial(_fin_body, act=act),
        grid=(N // BN,),
        in_specs=[pl.BlockSpec((NC, BN, PF), lambda i: (0, i, 0)),
                  pl.BlockSpec((BN, Fo), lambda i: (i, 0)),
                  pl.BlockSpec((1, Fo), lambda i: (0, 0))],
        out_specs=pl.BlockSpec((BN, Fo), lambda i: (i, 0)),
        out_shape=jax.ShapeDtypeStruct((N, Fo), F32),
    )(m2, so, b_row)


def _mesh():
    return plsc.VectorSubcoreMesh(core_axis_name="c", subcore_axis_name="s")


_SC_PARAMS = pltpu.CompilerParams(needs_layout_passes=False)


def _k1(F, xn, row, col, zn):
    @functools.partial(
        pl.kernel,
        out_type=[jax.ShapeDtypeStruct((E,), F32),
                  jax.ShapeDtypeStruct((NC, N), F32),
                  jax.ShapeDtypeStruct((NC, N), F32)],
        mesh=_mesh(),
        compiler_params=_SC_PARAMS,
        scratch_types=[
            pltpu.VMEM((C,), I32),
            pltpu.VMEM((C,), I32),
            pltpu.VMEM((C, PF), F32),
            pltpu.VMEM((C, PF), F32),
            pltpu.VMEM((C,), F32),
            pltpu.VMEM((C,), F32),
            pltpu.VMEM_SHARED((N,), F32),
            pltpu.VMEM_SHARED((N,), F32),
            pltpu.SemaphoreType.DMA,
            pltpu.SemaphoreType.DMA,
        ],
    )
    def body(xn_hbm, row_hbm, col_hbm, zn_hbm, simt_hbm, rs_hbm, dg_hbm,
             ridx, cidx, rowsA, rowsB, simv, indv, rs_sh, dg_sh, semA, semB):
        cid = lax.axis_index("c")
        sid = lax.axis_index("s")
        wid = sid * NC + cid

        @pl.when(sid == 0)
        def _():
            pltpu.sync_copy(zn_hbm, rs_sh)
            pltpu.sync_copy(zn_hbm, dg_sh)

        plsc.subcore_barrier()

        def chunk(i, carry):
            base = wid * EPW + i * C
            pltpu.sync_copy(row_hbm.at[pl.ds(base, C)], ridx)
            pltpu.sync_copy(col_hbm.at[pl.ds(base, C)], cidx)
            cpA = pltpu.async_copy(xn_hbm.at[ridx], rowsA, semA)
            cpB = pltpu.async_copy(xn_hbm.at[cidx], rowsB, semB)
            cpA.wait()
            cpB.wait()

            def group(g, c2):
                e0 = g * 16
                ei = jnp.arange(16, dtype=I32) + e0
                acc = jnp.zeros((16,), F32)
                for j in range(F):
                    jj = jnp.full((16,), j, I32)
                    a = plsc.load_gather(rowsA, [ei, jj])
                    b = plsc.load_gather(rowsB, [ei, jj])
                    acc = acc + a * b
                st = jnp.where(acc < P0, 0.0, acc)
                simv[pl.ds(e0, 16)] = st
                indv[pl.ds(e0, 16)] = jnp.where(st > 0.0, 1.0, 0.0)
                return c2

            lax.fori_loop(0, G, group, 0)
            pltpu.sync_copy(simv, simt_hbm.at[pl.ds(base, C)])
            pltpu.sync_copy(simv, rs_sh.at[ridx], add=True)
            pltpu.sync_copy(indv, dg_sh.at[ridx], add=True)
            return carry

        lax.fori_loop(0, NCH, chunk, 0)
        plsc.subcore_barrier()

        @pl.when(sid == 0)
        def _():
            pltpu.sync_copy(rs_sh, rs_hbm.at[cid])
            pltpu.sync_copy(dg_sh, dg_hbm.at[cid])

    return body(xn, row, col, zn)


def _k2(simt, row, col, denom, zn):
    @functools.partial(
        pl.kernel,
        out_type=[jax.ShapeDtypeStruct((E,), F32),
                  jax.ShapeDtypeStruct((NC, N), F32)],
        mesh=_mesh(),
        compiler_params=_SC_PARAMS,
        scratch_types=[
            pltpu.VMEM((C,), I32),
            pltpu.VMEM((C,), I32),
            pltpu.VMEM((C,), F32),
            pltpu.VMEM((C,), F32),
            pltpu.VMEM((N,), F32),
            pltpu.VMEM_SHARED((N,), F32),
        ],
    )
    def body(simt_hbm, row_hbm, col_hbm, den_hbm, zn_hbm, ew_hbm, dc_hbm,
             ridx, cidx, sv, ewv, denv, dc_sh):
        cid = lax.axis_index("c")
        sid = lax.axis_index("s")
        wid = sid * NC + cid

        pltpu.sync_copy(den_hbm, denv)

        @pl.when(sid == 0)
        def _():
            pltpu.sync_copy(zn_hbm, dc_sh)

        plsc.subcore_barrier()

        def chunk(i, carry):
            base = wid * EPW + i * C
            pltpu.sync_copy(row_hbm.at[pl.ds(base, C)], ridx)
            pltpu.sync_copy(col_hbm.at[pl.ds(base, C)], cidx)
            pltpu.sync_copy(simt_hbm.at[pl.ds(base, C)], sv)

            def group(g, c2):
                e0 = g * 16
                rv = ridx[pl.ds(e0, 16)]
                s = sv[pl.ds(e0, 16)]
                d = plsc.load_gather(denv, [rv])
                w = s / d
                ewv[pl.ds(e0, 16)] = jnp.where(s > 0.0, jnp.exp(w), 0.0)
                return c2

            lax.fori_loop(0, G, group, 0)
            pltpu.sync_copy(ewv, ew_hbm.at[pl.ds(base, C)])
            pltpu.sync_copy(ewv, dc_sh.at[cidx], add=True)
            return carry

        lax.fori_loop(0, NCH, chunk, 0)
        plsc.subcore_barrier()

        @pl.when(sid == 0)
        def _():
            pltpu.sync_copy(dc_sh, dc_hbm.at[cid])

    return body(simt, row, col, denom, zn)


def _k3(Fo, hws, ew, row, col, dis, znf):
    @functools.partial(
        pl.kernel,
        out_type=jax.ShapeDtypeStruct((NC, N, PF), F32),
        mesh=_mesh(),
        compiler_params=_SC_PARAMS,
        scratch_types=[
            pltpu.VMEM((C,), I32),
            pltpu.VMEM((C,), I32),
            pltpu.VMEM((C, PF), F32),
            pltpu.VMEM((C,), F32),
            pltpu.VMEM((N,), F32),
            pltpu.VMEM_SHARED((N, PF), F32),
            pltpu.SemaphoreType.DMA,
        ],
    )
    def body(hws_hbm, ew_hbm, row_hbm, col_hbm, dis_hbm, znf_hbm, msg_hbm,
             ridx, cidx, rows, ewv, disv, out_sh, semA):
        cid = lax.axis_index("c")
        sid = lax.axis_index("s")
        wid = sid * NC + cid

        pltpu.sync_copy(dis_hbm, disv)

        @pl.when(sid == 0)
        def _():
            pltpu.sync_copy(znf_hbm, out_sh)

        plsc.subcore_barrier()

        def chunk(i, carry):
            base = wid * EPW + i * C
            pltpu.sync_copy(row_hbm.at[pl.ds(base, C)], ridx)
            pltpu.sync_copy(col_hbm.at[pl.ds(base, C)], cidx)
            pltpu.sync_copy(ew_hbm.at[pl.ds(base, C)], ewv)
            pltpu.async_copy(hws_hbm.at[ridx], rows, semA).wait()

            def group(g, c2):
                e0 = g * 16
                ei = jnp.arange(16, dtype=I32) + e0
                cv = cidx[pl.ds(e0, 16)]
                sc = ewv[pl.ds(e0, 16)] * plsc.load_gather(disv, [cv])
                for j in range(Fo):
                    jj = jnp.full((16,), j, I32)
                    v = plsc.load_gather(rows, [ei, jj]) * sc
                    plsc.store_scatter(rows, [ei, jj], v)
                return c2

            lax.fori_loop(0, G, group, 0)
            pltpu.sync_copy(rows, out_sh.at[cidx], add=True)
            return carry

        lax.fori_loop(0, NCH, chunk, 0)
        plsc.subcore_barrier()

        @pl.when(sid == 0)
        def _():
            pltpu.sync_copy(out_sh, msg_hbm.at[cid])

    return body(hws, ew, row, col, dis, znf)


def kernel(x, adj_indices, W1, b1, W2, b2, W3, b3):
    row, col = adj_indices[0], adj_indices[1]
    zn = jnp.zeros((N,), F32)
    h = x
    for li, (W, b) in enumerate(((W1, b1), (W2, b2), (W3, b3))):
        F, Fo = h.shape[1], W.shape[1]
        xn, hw = _prep(h, W)
        simt, rs2, dg2 = _k1(F, xn, row, col, zn)
        den, slw = _mid(rs2, dg2)
        ew, dc2 = _k2(simt, row, col, den.reshape(N), zn)
        dis = _post1(dc2, slw)
        hws, so = _post2(hw, dis.reshape(N, 1), slw.reshape(N, 1))
        znf = jnp.zeros((N, PF), F32)
        m2 = _k3(Fo, hws, ew, row, col, dis.reshape(N), znf)
        h = _fin(m2, so, b.reshape(1, Fo), "lsm" if li == 2 else "relu")
    return h


# speedup vs baseline: 7.1824x; 1.2690x over previous
import functools

import jax
import jax.numpy as jnp
from jax import lax
from jax.experimental import pallas as pl
from jax.experimental.pallas import tpu as pltpu
from jax.experimental.pallas import tpu_sc as plsc

N = 10000
E = 320000
P0 = 0.5
NC = 2
NS = 16
NW = NC * NS
EPW = E // NW
C = 80
G = C // 16
NCH = EPW // C

PF = 128

F32 = jnp.float32
I32 = jnp.int32


def _prep_body(h_ref, w_ref, xn_ref, hw_ref):
    h = h_ref[...]
    n2 = jnp.sum(h * h, axis=1, keepdims=True)
    safe = jnp.where(n2 == 0.0, 1.0, jnp.sqrt(n2))
    xn = h / safe
    F = h.shape[1]
    if F < PF:
        xn = jnp.concatenate([xn, jnp.zeros((h.shape[0], PF - F), F32)], axis=1)
    xn_ref[...] = xn
    hw_ref[...] = jnp.dot(h, w_ref[...], preferred_element_type=F32)


def _prep(h, W):
    F, Fo = h.shape[1], W.shape[1]
    BN = 2000
    return pl.pallas_call(
        _prep_body,
        grid=(N // BN,),
        in_specs=[pl.BlockSpec((BN, F), lambda i: (i, 0)),
                  pl.BlockSpec((F, Fo), lambda i: (0, 0))],
        out_specs=[pl.BlockSpec((BN, PF), lambda i: (i, 0)),
                   pl.BlockSpec((BN, Fo), lambda i: (i, 0))],
        out_shape=[jax.ShapeDtypeStruct((N, PF), F32),
                   jax.ShapeDtypeStruct((N, Fo), F32)],
    )(h, W)


def _mid_body(rs_ref, dg_ref, den_ref, sl_ref):
    rs = jnp.sum(rs_ref[...], axis=0, keepdims=True)
    dg = jnp.sum(dg_ref[...], axis=0, keepdims=True)
    den_ref[...] = jnp.where(rs == 0.0, 1.0, rs)
    sl_ref[...] = jnp.exp(1.0 / (dg + 1.0))


def _mid(rs2, dg2):
    return pl.pallas_call(
        _mid_body,
        out_shape=[jax.ShapeDtypeStruct((1, N), F32),
                   jax.ShapeDtypeStruct((1, N), F32)],
    )(rs2, dg2)


def _post1_body(dc_ref, sl_ref, dis_ref):
    degc = jnp.sum(dc_ref[...], axis=0, keepdims=True) + sl_ref[...]
    pos = degc > 0.0
    dis_ref[...] = jnp.where(pos, lax.rsqrt(jnp.where(pos, degc, 1.0)), 0.0)


def _post1(dc2, slw):
    return pl.pallas_call(
        _post1_body,
        out_shape=jax.ShapeDtypeStruct((1, N), F32),
    )(dc2, slw)


def _post2_body(hw_ref, dis_ref, sl_ref, hws_ref, so_ref):
    hw = hw_ref[...]
    dis = dis_ref[...]
    hws = hw * dis
    Fo = hw.shape[1]
    if Fo < PF:
        hws = jnp.concatenate(
            [hws, jnp.zeros((hw.shape[0], PF - Fo), F32)], axis=1)
    hws_ref[...] = hws
    so_ref[...] = hw * (dis * dis * sl_ref[...])


def _post2(hw, dis_col, sl_col):
    Fo = hw.shape[1]
    BN = 2000
    return pl.pallas_call(
        _post2_body,
        grid=(N // BN,),
        in_specs=[pl.BlockSpec((BN, Fo), lambda i: (i, 0)),
                  pl.BlockSpec((BN, 1), lambda i: (i, 0)),
                  pl.BlockSpec((BN, 1), lambda i: (i, 0))],
        out_specs=[pl.BlockSpec((BN, PF), lambda i: (i, 0)),
                   pl.BlockSpec((BN, Fo), lambda i: (i, 0))],
        out_shape=[jax.ShapeDtypeStruct((N, PF), F32),
                   jax.ShapeDtypeStruct((N, Fo), F32)],
    )(hw, dis_col, sl_col)


def _fin_body(m_ref, so_ref, b_ref, o_ref, *, act):
    Fo = so_ref.shape[1]
    m = m_ref[...]
    s = m[0, :, :Fo] + m[1, :, :Fo] + so_ref[...] + b_ref[...]
    if act == "relu":
        o_ref[...] = jnp.maximum(s, 0.0)
    elif act == "lsm":
        mx = jnp.max(s, axis=1, keepdims=True)
        z = s - mx
        o_ref[...] = z - jnp.log(jnp.sum(jnp.exp(z), axis=1, keepdims=True))
    else:
        o_ref[...] = s


def _fin(m2, so, b_row, act):
    Fo = so.shape[1]
    BN = 2000
    return pl.pallas_call(
        functools.partial(_fin_body, act=act),
        grid=(N // BN,),
        in_specs=[pl.BlockSpec((NC, BN, PF), lambda i: (0, i, 0)),
                  pl.BlockSpec((BN, Fo), lambda i: (i, 0)),
                  pl.BlockSpec((1, Fo), lambda i: (0, 0))],
        out_specs=pl.BlockSpec((BN, Fo), lambda i: (i, 0)),
        out_shape=jax.ShapeDtypeStruct((N, Fo), F32),
    )(m2, so, b_row)


def _mesh():
    return plsc.VectorSubcoreMesh(core_axis_name="c", subcore_axis_name="s")


_SC_PARAMS = pltpu.CompilerParams(needs_layout_passes=False)


def _k1(F, xn, row, col, zn):
    @functools.partial(
        pl.kernel,
        out_type=[jax.ShapeDtypeStruct((E,), F32),
                  jax.ShapeDtypeStruct((NC, N), F32),
                  jax.ShapeDtypeStruct((NC, N), F32)],
        mesh=_mesh(),
        compiler_params=_SC_PARAMS,
        scratch_types=[
            pltpu.VMEM((C,), I32),
            pltpu.VMEM((C,), I32),
            pltpu.VMEM((C, PF), F32),
            pltpu.VMEM((C, PF), F32),
            pltpu.VMEM((C,), F32),
            pltpu.VMEM((C,), F32),
            pltpu.VMEM_SHARED((N,), F32),
            pltpu.VMEM_SHARED((N,), F32),
            pltpu.SemaphoreType.DMA,
            pltpu.SemaphoreType.DMA,
        ],
    )
    def body(xn_hbm, row_hbm, col_hbm, zn_hbm, simt_hbm, rs_hbm, dg_hbm,
             ridx, cidx, rowsA, rowsB, simv, indv, rs_sh, dg_sh, semA, semB):
        cid = lax.axis_index("c")
        sid = lax.axis_index("s")
        wid = sid * NC + cid

        @pl.when(sid == 0)
        def _():
            pltpu.sync_copy(zn_hbm, rs_sh)
            pltpu.sync_copy(zn_hbm, dg_sh)

        plsc.subcore_barrier()

        def chunk(i, carry):
            base = wid * EPW + i * C
            pltpu.sync_copy(row_hbm.at[pl.ds(base, C)], ridx)
            pltpu.sync_copy(col_hbm.at[pl.ds(base, C)], cidx)
            cpA = pltpu.async_copy(xn_hbm.at[ridx], rowsA, semA)
            cpB = pltpu.async_copy(xn_hbm.at[cidx], rowsB, semB)
            cpA.wait()
            cpB.wait()

            def group(g, c2):
                e0 = g * 16
                ei = jnp.arange(16, dtype=I32) + e0
                na = 4
                accs = [jnp.zeros((16,), F32) for _ in range(na)]
                for j in range(F):
                    jj = jnp.full((16,), j, I32)
                    a = plsc.load_gather(rowsA, [ei, jj])
                    b = plsc.load_gather(rowsB, [ei, jj])
                    accs[j % na] = accs[j % na] + a * b
                acc = (accs[0] + accs[1]) + (accs[2] + accs[3])
                st = jnp.where(acc < P0, 0.0, acc)
                simv[pl.ds(e0, 16)] = st
                indv[pl.ds(e0, 16)] = jnp.where(st > 0.0, 1.0, 0.0)
                return c2

            lax.fori_loop(0, G, group, 0)
            pltpu.sync_copy(simv, simt_hbm.at[pl.ds(base, C)])
            pltpu.sync_copy(simv, rs_sh.at[ridx], add=True)
            pltpu.sync_copy(indv, dg_sh.at[ridx], add=True)
            return carry

        lax.fori_loop(0, NCH, chunk, 0)
        plsc.subcore_barrier()

        @pl.when(sid == 0)
        def _():
            pltpu.sync_copy(rs_sh, rs_hbm.at[cid])
            pltpu.sync_copy(dg_sh, dg_hbm.at[cid])

    return body(xn, row, col, zn)


def _k2(simt, row, col, denom, zn):
    @functools.partial(
        pl.kernel,
        out_type=[jax.ShapeDtypeStruct((E,), F32),
                  jax.ShapeDtypeStruct((NC, N), F32)],
        mesh=_mesh(),
        compiler_params=_SC_PARAMS,
        scratch_types=[
            pltpu.VMEM((C,), I32),
            pltpu.VMEM((C,), I32),
            pltpu.VMEM((C,), F32),
            pltpu.VMEM((C,), F32),
            pltpu.VMEM((N,), F32),
            pltpu.VMEM_SHARED((N,), F32),
        ],
    )
    def body(simt_hbm, row_hbm, col_hbm, den_hbm, zn_hbm, ew_hbm, dc_hbm,
             ridx, cidx, sv, ewv, denv, dc_sh):
        cid = lax.axis_index("c")
        sid = lax.axis_index("s")
        wid = sid * NC + cid

        pltpu.sync_copy(den_hbm, denv)

        @pl.when(sid == 0)
        def _():
            pltpu.sync_copy(zn_hbm, dc_sh)

        plsc.subcore_barrier()

        def chunk(i, carry):
            base = wid * EPW + i * C
            pltpu.sync_copy(row_hbm.at[pl.ds(base, C)], ridx)
            pltpu.sync_copy(col_hbm.at[pl.ds(base, C)], cidx)
            pltpu.sync_copy(simt_hbm.at[pl.ds(base, C)], sv)

            def group(g, c2):
                e0 = g * 16
                rv = ridx[pl.ds(e0, 16)]
                s = sv[pl.ds(e0, 16)]
                d = plsc.load_gather(denv, [rv])
                w = s / d
                ewv[pl.ds(e0, 16)] = jnp.where(s > 0.0, jnp.exp(w), 0.0)
                return c2

            lax.fori_loop(0, G, group, 0)
            pltpu.sync_copy(ewv, ew_hbm.at[pl.ds(base, C)])
            pltpu.sync_copy(ewv, dc_sh.at[cidx], add=True)
            return carry

        lax.fori_loop(0, NCH, chunk, 0)
        plsc.subcore_barrier()

        @pl.when(sid == 0)
        def _():
            pltpu.sync_copy(dc_sh, dc_hbm.at[cid])

    return body(simt, row, col, denom, zn)


def _k3(Fo, hws, ew, row, col, dis, znf):
    @functools.partial(
        pl.kernel,
        out_type=jax.ShapeDtypeStruct((NC, N, PF), F32),
        mesh=_mesh(),
        compiler_params=_SC_PARAMS,
        scratch_types=[
            pltpu.VMEM((C,), I32),
            pltpu.VMEM((C,), I32),
            pltpu.VMEM((C, PF), F32),
            pltpu.VMEM((C,), F32),
            pltpu.VMEM((C,), F32),
            pltpu.VMEM((N,), F32),
            pltpu.VMEM_SHARED((N, PF), F32),
            pltpu.SemaphoreType.DMA,
        ],
    )
    def body(hws_hbm, ew_hbm, row_hbm, col_hbm, dis_hbm, znf_hbm, msg_hbm,
             ridx, cidx, rows, ewv, scalev, disv, out_sh, semA):
        cid = lax.axis_index("c")
        sid = lax.axis_index("s")
        wid = sid * NC + cid

        pltpu.sync_copy(dis_hbm, disv)

        @pl.when(sid == 0)
        def _():
            pltpu.sync_copy(znf_hbm, out_sh)

        plsc.subcore_barrier()

        def chunk(i, carry):
            base = wid * EPW + i * C
            pltpu.sync_copy(row_hbm.at[pl.ds(base, C)], ridx)
            pltpu.sync_copy(col_hbm.at[pl.ds(base, C)], cidx)
            pltpu.sync_copy(ew_hbm.at[pl.ds(base, C)], ewv)
            pltpu.async_copy(hws_hbm.at[ridx], rows, semA).wait()

            def group(g, c2):
                e0 = g * 16
                cv = cidx[pl.ds(e0, 16)]
                sc = ewv[pl.ds(e0, 16)] * plsc.load_gather(disv, [cv])
                scalev[pl.ds(e0, 16)] = sc
                return c2

            lax.fori_loop(0, G, group, 0)
            nsl = -(-Fo // 16)

            def edge(e, c2):
                sc = plsc.load_gather(scalev, [jnp.full((16,), e, I32)])
                r = rows.at[e]
                for k in range(nsl):
                    r[pl.ds(k * 16, 16)] = r[pl.ds(k * 16, 16)] * sc
                return c2

            lax.fori_loop(0, C, edge, 0)
            pltpu.sync_copy(rows, out_sh.at[cidx], add=True)
            return carry

        lax.fori_loop(0, NCH, chunk, 0)
        plsc.subcore_barrier()

        @pl.when(sid == 0)
        def _():
            pltpu.sync_copy(out_sh, msg_hbm.at[cid])

    return body(hws, ew, row, col, dis, znf)


def kernel(x, adj_indices, W1, b1, W2, b2, W3, b3):
    row, col = adj_indices[0], adj_indices[1]
    zn = jnp.zeros((N,), F32)
    h = x
    for li, (W, b) in enumerate(((W1, b1), (W2, b2), (W3, b3))):
        F, Fo = h.shape[1], W.shape[1]
        xn, hw = _prep(h, W)
        simt, rs2, dg2 = _k1(F, xn, row, col, zn)
        den, slw = _mid(rs2, dg2)
        ew, dc2 = _k2(simt, row, col, den.reshape(N), zn)
        dis = _post1(dc2, slw)
        hws, so = _post2(hw, dis.reshape(N, 1), slw.reshape(N, 1))
        znf = jnp.zeros((N, PF), F32)
        m2 = _k3(Fo, hws, ew, row, col, dis.reshape(N), znf)
        h = _fin(m2, so, b.reshape(1, Fo), "lsm" if li == 2 else "relu")
    return h


# speedup vs baseline: 16.9455x; 2.3593x over previous
import functools

import jax
import jax.numpy as jnp
from jax import lax
from jax.experimental import pallas as pl
from jax.experimental.pallas import tpu as pltpu
from jax.experimental.pallas import tpu_sc as plsc

N = 10000
E = 320000
P0 = 0.5
NC = 2
NS = 16
NW = NC * NS
EPW = E // NW
C = 128
G = C // 16
NCHP = 84
NCH2 = NCHP // 2
NCH3 = NCHP // 2
EPAD = NCHP * C
EP = NW * EPAD
PADIN = (NW - 1) * EPW + EPAD - E
PF = 128

F32 = jnp.float32
I32 = jnp.int32


def _prep_body(h_ref, w_ref, xn_ref, hw_ref):
    h = h_ref[...]
    n2 = jnp.sum(h * h, axis=1, keepdims=True)
    safe = jnp.where(n2 == 0.0, 1.0, jnp.sqrt(n2))
    xn = h / safe
    F = h.shape[1]
    if F < PF:
        xn = jnp.concatenate([xn, jnp.zeros((h.shape[0], PF - F), F32)], axis=1)
    xn_ref[...] = xn
    hw_ref[...] = jnp.dot(h, w_ref[...], preferred_element_type=F32)


def _prep(h, W):
    F, Fo = h.shape[1], W.shape[1]
    BN = 2000
    return pl.pallas_call(
        _prep_body,
        grid=(N // BN,),
        in_specs=[pl.BlockSpec((BN, F), lambda i: (i, 0)),
                  pl.BlockSpec((F, Fo), lambda i: (0, 0))],
        out_specs=[pl.BlockSpec((BN, PF), lambda i: (i, 0)),
                   pl.BlockSpec((BN, Fo), lambda i: (i, 0))],
        out_shape=[jax.ShapeDtypeStruct((N, PF), F32),
                   jax.ShapeDtypeStruct((N, Fo), F32)],
    )(h, W)


def _mid_body(rs_ref, dg_ref, den_ref, sl_ref):
    rs = jnp.sum(rs_ref[...], axis=0, keepdims=True)
    dg = jnp.sum(dg_ref[...], axis=0, keepdims=True)
    den_ref[...] = jnp.where(rs == 0.0, 1.0, rs)
    sl_ref[...] = jnp.exp(1.0 / (dg + 1.0))


def _mid(rs2, dg2):
    return pl.pallas_call(
        _mid_body,
        out_shape=[jax.ShapeDtypeStruct((1, N), F32),
                   jax.ShapeDtypeStruct((1, N), F32)],
    )(rs2, dg2)


def _post1_body(dc_ref, sl_ref, dis_ref):
    degc = jnp.sum(dc_ref[...], axis=0, keepdims=True) + sl_ref[...]
    pos = degc > 0.0
    dis_ref[...] = jnp.where(pos, lax.rsqrt(jnp.where(pos, degc, 1.0)), 0.0)


def _post1(dc2, slw):
    return pl.pallas_call(
        _post1_body,
        out_shape=jax.ShapeDtypeStruct((1, N), F32),
    )(dc2, slw)


def _post2_body(hw_ref, dis_ref, sl_ref, hws_ref, so_ref):
    hw = hw_ref[...]
    dis = dis_ref[...]
    hws = hw * dis
    Fo = hw.shape[1]
    if Fo < PF:
        hws = jnp.concatenate(
            [hws, jnp.zeros((hw.shape[0], PF - Fo), F32)], axis=1)
    hws_ref[...] = hws
    so_ref[...] = hw * (dis * dis * sl_ref[...])


def _post2(hw, dis_col, sl_col):
    Fo = hw.shape[1]
    BN = 2000
    return pl.pallas_call(
        _post2_body,
        grid=(N // BN,),
        in_specs=[pl.BlockSpec((BN, Fo), lambda i: (i, 0)),
                  pl.BlockSpec((BN, 1), lambda i: (i, 0)),
                  pl.BlockSpec((BN, 1), lambda i: (i, 0))],
        out_specs=[pl.BlockSpec((BN, PF), lambda i: (i, 0)),
                   pl.BlockSpec((BN, Fo), lambda i: (i, 0))],
        out_shape=[jax.ShapeDtypeStruct((N, PF), F32),
                   jax.ShapeDtypeStruct((N, Fo), F32)],
    )(hw, dis_col, sl_col)


def _fin_body(m_ref, so_ref, b_ref, o_ref, *, act):
    Fo = so_ref.shape[1]
    m = m_ref[...]
    s = m[0, :, :Fo] + m[1, :, :Fo] + so_ref[...] + b_ref[...]
    if act == "relu":
        o_ref[...] = jnp.maximum(s, 0.0)
    elif act == "lsm":
        mx = jnp.max(s, axis=1, keepdims=True)
        z = s - mx
        o_ref[...] = z - jnp.log(jnp.sum(jnp.exp(z), axis=1, keepdims=True))
    else:
        o_ref[...] = s


def _fin(m2, so, b_row, act):
    Fo = so.shape[1]
    BN = 2000
    return pl.pallas_call(
        functools.partial(_fin_body, act=act),
        grid=(N // BN,),
        in_specs=[pl.BlockSpec((NC, BN, PF), lambda i: (0, i, 0)),
                  pl.BlockSpec((BN, Fo), lambda i: (i, 0)),
                  pl.BlockSpec((1, Fo), lambda i: (0, 0))],
        out_specs=pl.BlockSpec((BN, Fo), lambda i: (i, 0)),
        out_shape=jax.ShapeDtypeStruct((N, Fo), F32),
    )(m2, so, b_row)


def _mesh():
    return plsc.VectorSubcoreMesh(core_axis_name="c", subcore_axis_name="s")


_SC_PARAMS = pltpu.CompilerParams(needs_layout_passes=False)


def _copy_idx(src, dst):
    for q in range(C // 16):
        dst[pl.ds(q * 16, 16)] = src[pl.ds(q * 16, 16)]


def _k1(F, xn, row, col, zn):
    @functools.partial(
        pl.kernel,
        out_type=[jax.ShapeDtypeStruct((EP,), F32),
                  jax.ShapeDtypeStruct((NC, N), F32),
                  jax.ShapeDtypeStruct((NC, N), F32)],
        mesh=_mesh(),
        compiler_params=_SC_PARAMS,
        scratch_types=[
            pltpu.VMEM((C,), I32), pltpu.VMEM((C,), I32),
            pltpu.VMEM((C, PF), F32), pltpu.VMEM((C, PF), F32),
            pltpu.VMEM((C,), F32), pltpu.VMEM((C,), F32),
            pltpu.VMEM_SHARED((N,), F32),
            pltpu.VMEM_SHARED((N,), F32),
            pltpu.SemaphoreType.DMA, pltpu.SemaphoreType.DMA,
        ],
    )
    def body(xn_hbm, row_hbm, col_hbm, zn_hbm, simt_hbm, rs_hbm, dg_hbm,
             ridx0, cidx0,
             rowsA0, rowsB0,
             simv0, indv0,
             rs_sh, dg_sh, semG0, semG1):
        cid = lax.axis_index("c")
        sid = lax.axis_index("s")
        wid = sid * NC + cid

        @pl.when(sid == 0)
        def _():
            pltpu.sync_copy(zn_hbm, rs_sh)
            pltpu.sync_copy(zn_hbm, dg_sh)

        plsc.subcore_barrier()

        def in_base(k):
            return wid * EPW + k * C

        def p_base(k):
            return wid * EPAD + k * C

        def load_idx(k, r, c):
            c1 = pltpu.async_copy(row_hbm.at[pl.ds(in_base(k), C)], r, semG0)
            c2 = pltpu.async_copy(col_hbm.at[pl.ds(in_base(k), C)], c, semG1)
            c1.wait()
            c2.wait()

        na = 8 if F % 8 == 0 else 4
        lane = jnp.arange(16, dtype=I32)

        def compute(k, rA, rB, sv, iv):
            def group(g, c2):
                e0 = g * 16
                ei = lane + e0
                zero = jnp.zeros((16,), F32)

                def jstep(jv, accs):
                    res = list(accs)
                    j0 = jv * na
                    for t in range(na):
                        jj = (lane + (j0 + t)) % F
                        a = plsc.load_gather(rA, [ei, jj])
                        b = plsc.load_gather(rB, [ei, jj])
                        res[t] = res[t] + a * b
                    return tuple(res)

                accs = lax.fori_loop(0, F // na, jstep,
                                     tuple(zero for _ in range(na)))
                acc = accs[0]
                for t in range(1, na):
                    acc = acc + accs[t]
                valid = k * C + e0 < EPW
                st = jnp.where(valid & (acc >= P0), acc, 0.0)
                sv[pl.ds(e0, 16)] = st
                iv[pl.ds(e0, 16)] = jnp.where(st > 0.0, 1.0, 0.0)
                return c2

            lax.fori_loop(0, G, group, 0)

        def chunk(i, carry):
            load_idx(i, ridx0, cidx0)
            cpA = pltpu.async_copy(xn_hbm.at[ridx0], rowsA0, semG0)
            cpB = pltpu.async_copy(xn_hbm.at[cidx0], rowsB0, semG1)
            cpA.wait()
            cpB.wait()
            compute(i, rowsA0, rowsB0, simv0, indv0)
            pltpu.sync_copy(simv0, simt_hbm.at[pl.ds(p_base(i), C)])
            pltpu.sync_copy(simv0, rs_sh.at[ridx0], add=True)
            pltpu.sync_copy(indv0, dg_sh.at[ridx0], add=True)
            return carry

        lax.fori_loop(0, NCHP, chunk, 0)
        plsc.subcore_barrier()

        @pl.when(sid == 0)
        def _():
            pltpu.sync_copy(rs_sh, rs_hbm.at[cid])
            pltpu.sync_copy(dg_sh, dg_hbm.at[cid])

    return body(xn, row, col, zn)


def _k2(simt, row, col, denom, zn):
    @functools.partial(
        pl.kernel,
        out_type=[jax.ShapeDtypeStruct((EP,), F32),
                  jax.ShapeDtypeStruct((NC, N), F32)],
        mesh=_mesh(),
        compiler_params=_SC_PARAMS,
        scratch_types=[
            pltpu.VMEM((C,), I32),
            pltpu.VMEM((C,), I32),
            pltpu.VMEM((C,), F32),
            pltpu.VMEM((C,), F32),
            pltpu.VMEM((N,), F32),
            pltpu.VMEM_SHARED((N,), F32),
            pltpu.SemaphoreType.DMA,
        ],
    )
    def body(simt_hbm, row_hbm, col_hbm, den_hbm, zn_hbm, ew_hbm, dc_hbm,
             ridx, cidx, sv, ewv, denv, dc_sh, semI):
        cid = lax.axis_index("c")
        sid = lax.axis_index("s")
        wid = sid * NC + cid

        pltpu.sync_copy(den_hbm, denv)

        @pl.when(sid == 0)
        def _():
            pltpu.sync_copy(zn_hbm, dc_sh)

        plsc.subcore_barrier()

        def chunk(i, carry):
            base = wid * EPW + i * C
            basep = wid * EPAD + i * C
            c1 = pltpu.async_copy(row_hbm.at[pl.ds(base, C)], ridx, semI)
            c2 = pltpu.async_copy(col_hbm.at[pl.ds(base, C)], cidx, semI)
            c3 = pltpu.async_copy(simt_hbm.at[pl.ds(basep, C)], sv, semI)
            c1.wait()
            c2.wait()
            c3.wait()

            def group(g, c2_):
                e0 = g * 16
                rv = ridx[pl.ds(e0, 16)]
                s = sv[pl.ds(e0, 16)]
                d = plsc.load_gather(denv, [rv])
                w = s / d
                ewv[pl.ds(e0, 16)] = jnp.where(s > 0.0, jnp.exp(w), 0.0)
                return c2_

            lax.fori_loop(0, G, group, 0)
            pltpu.sync_copy(ewv, ew_hbm.at[pl.ds(basep, C)])
            pltpu.sync_copy(ewv, dc_sh.at[cidx], add=True)
            return carry

        lax.fori_loop(0, NCHP, chunk, 0)
        plsc.subcore_barrier()

        @pl.when(sid == 0)
        def _():
            pltpu.sync_copy(dc_sh, dc_hbm.at[cid])

    return body(simt, row, col, denom, zn)


def _k3(Fo, hws, ew, row, col, dis, znf):
    @functools.partial(
        pl.kernel,
        out_type=jax.ShapeDtypeStruct((NC, N, PF), F32),
        mesh=_mesh(),
        compiler_params=_SC_PARAMS,
        scratch_types=[
            pltpu.VMEM((C,), I32), pltpu.VMEM((C,), I32),
            pltpu.VMEM((C,), I32), pltpu.VMEM((C,), I32),
            pltpu.VMEM((C,), I32), pltpu.VMEM((C,), I32),
            pltpu.VMEM((C, PF), F32), pltpu.VMEM((C, PF), F32),
            pltpu.VMEM((C,), F32), pltpu.VMEM((C,), F32),
            pltpu.VMEM((C,), F32),
            pltpu.VMEM((N,), F32),
            pltpu.VMEM_SHARED((N, PF), F32),
            pltpu.SemaphoreType.DMA,
            pltpu.SemaphoreType.DMA, pltpu.SemaphoreType.DMA,
            pltpu.SemaphoreType.DMA, pltpu.SemaphoreType.DMA,
        ],
    )
    def body(hws_hbm, ew_hbm, row_hbm, col_hbm, dis_hbm, znf_hbm, msg_hbm,
             ridx0, cidx0, ridx1, cidx1,
             cidxS0, cidxS1,
             rows0, rows1,
             ewv0, ewv1, scalev,
             disv, out_sh, semI, semG0, semG1, semO0, semO1):
        cid = lax.axis_index("c")
        sid = lax.axis_index("s")
        wid = sid * NC + cid

        pltpu.sync_copy(dis_hbm, disv)

        @pl.when(sid == 0)
        def _():
            pltpu.sync_copy(znf_hbm, out_sh)

        plsc.subcore_barrier()

        def in_base(k):
            return wid * EPW + k * C

        def p_base(k):
            return wid * EPAD + k * C

        def load_in(k, r, c, e):
            c1 = pltpu.async_copy(row_hbm.at[pl.ds(in_base(k), C)], r, semI)
            c2 = pltpu.async_copy(col_hbm.at[pl.ds(in_base(k), C)], c, semI)
            c3 = pltpu.async_copy(ew_hbm.at[pl.ds(p_base(k), C)], e, semI)
            c1.wait()
            c2.wait()
            c3.wait()

        nsl = -(-Fo // 16)

        def compute(r_rows, cidx, ewv, cidxS):
            def group(g, c2_):
                e0 = g * 16
                cv = cidx[pl.ds(e0, 16)]
                sc = ewv[pl.ds(e0, 16)] * plsc.load_gather(disv, [cv])
                scalev[pl.ds(e0, 16)] = sc
                return c2_

            lax.fori_loop(0, G, group, 0)

            def edge(e, c2_):
                sc = plsc.load_gather(scalev, [jnp.full((16,), e, I32)])
                r = r_rows.at[e]
                for k in range(nsl):
                    r[pl.ds(k * 16, 16)] = r[pl.ds(k * 16, 16)] * sc
                return c2_

            lax.fori_loop(0, C, edge, 0)

        def chunk(i, carry):
            load_in(i, ridx0, cidx0, ewv0)
            pltpu.async_copy(hws_hbm.at[ridx0], rows0, semG0).wait()
            compute(rows0, cidx0, ewv0, cidxS0)
            pltpu.sync_copy(rows0, out_sh.at[cidx0], add=True)
            return carry

        lax.fori_loop(0, NCHP, chunk, 0)
        plsc.subcore_barrier()

        @pl.when(sid == 0)
        def _():
            pltpu.sync_copy(out_sh, msg_hbm.at[cid])

    return body(hws, ew, row, col, dis, znf)


def kernel(x, adj_indices, W1, b1, W2, b2, W3, b3):
    row = jnp.pad(adj_indices[0], (0, PADIN))
    col = jnp.pad(adj_indices[1], (0, PADIN))
    zn = jnp.zeros((N,), F32)
    h = x
    for li, (W, b) in enumerate(((W1, b1), (W2, b2), (W3, b3))):
        F, Fo = h.shape[1], W.shape[1]
        xn, hw = _prep(h, W)
        simt, rs2, dg2 = _k1(F, xn, row, col, zn)
        den, slw = _mid(rs2, dg2)
        ew, dc2 = _k2(simt, row, col, den.reshape(N), zn)
        dis = _post1(dc2, slw)
        hws, so = _post2(hw, dis.reshape(N, 1), slw.reshape(N, 1))
        znf = jnp.zeros((N, PF), F32)
        m2 = _k3(Fo, hws, ew, row, col, dis.reshape(N), znf)
        h = _fin(m2, so, b.reshape(1, Fo), "lsm" if li == 2 else "relu")
    return h


# speedup vs baseline: 19.5520x; 1.1538x over previous
import functools

import jax
import jax.numpy as jnp
from jax import lax
from jax.experimental import pallas as pl
from jax.experimental.pallas import tpu as pltpu
from jax.experimental.pallas import tpu_sc as plsc

N = 10000
E = 320000
P0 = 0.5
NC = 2
NS = 16
NW = NC * NS
EPW = E // NW
C = 128
G = C // 16
NCHP = 84
NCH2 = NCHP // 2
NCH3 = NCHP // 2
EPAD = NCHP * C
EP = NW * EPAD
PADIN = (NW - 1) * EPW + EPAD - E
PF = 128

F32 = jnp.float32
I32 = jnp.int32


def _prep_body(h_ref, w_ref, xn_ref, hw_ref):
    h = h_ref[...]
    n2 = jnp.sum(h * h, axis=1, keepdims=True)
    safe = jnp.where(n2 == 0.0, 1.0, jnp.sqrt(n2))
    xn = h / safe
    F = h.shape[1]
    if F < PF:
        xn = jnp.concatenate([xn, jnp.zeros((h.shape[0], PF - F), F32)], axis=1)
    xn_ref[...] = xn
    hw_ref[...] = jnp.dot(h, w_ref[...], preferred_element_type=F32)


def _prep(h, W):
    F, Fo = h.shape[1], W.shape[1]
    BN = 2000
    return pl.pallas_call(
        _prep_body,
        grid=(N // BN,),
        in_specs=[pl.BlockSpec((BN, F), lambda i: (i, 0)),
                  pl.BlockSpec((F, Fo), lambda i: (0, 0))],
        out_specs=[pl.BlockSpec((BN, PF), lambda i: (i, 0)),
                   pl.BlockSpec((BN, Fo), lambda i: (i, 0))],
        out_shape=[jax.ShapeDtypeStruct((N, PF), F32),
                   jax.ShapeDtypeStruct((N, Fo), F32)],
    )(h, W)


def _mid_body(rs_ref, dg_ref, den_ref, sl_ref):
    rs = jnp.sum(rs_ref[...], axis=0, keepdims=True)
    dg = jnp.sum(dg_ref[...], axis=0, keepdims=True)
    den_ref[...] = jnp.where(rs == 0.0, 1.0, rs)
    sl_ref[...] = jnp.exp(1.0 / (dg + 1.0))


def _mid(rs2, dg2):
    return pl.pallas_call(
        _mid_body,
        out_shape=[jax.ShapeDtypeStruct((1, N), F32),
                   jax.ShapeDtypeStruct((1, N), F32)],
    )(rs2, dg2)


def _post1_body(dc_ref, sl_ref, dis_ref):
    degc = jnp.sum(dc_ref[...], axis=0, keepdims=True) + sl_ref[...]
    pos = degc > 0.0
    dis_ref[...] = jnp.where(pos, lax.rsqrt(jnp.where(pos, degc, 1.0)), 0.0)


def _post1(dc2, slw):
    return pl.pallas_call(
        _post1_body,
        out_shape=jax.ShapeDtypeStruct((1, N), F32),
    )(dc2, slw)


def _post2_body(hw_ref, dis_ref, sl_ref, hws_ref, so_ref):
    hw = hw_ref[...]
    dis = dis_ref[...]
    hws = hw * dis
    Fo = hw.shape[1]
    if Fo < PF:
        hws = jnp.concatenate(
            [hws, jnp.zeros((hw.shape[0], PF - Fo), F32)], axis=1)
    hws_ref[...] = hws
    so_ref[...] = hw * (dis * dis * sl_ref[...])


def _post2(hw, dis_col, sl_col):
    Fo = hw.shape[1]
    BN = 2000
    return pl.pallas_call(
        _post2_body,
        grid=(N // BN,),
        in_specs=[pl.BlockSpec((BN, Fo), lambda i: (i, 0)),
                  pl.BlockSpec((BN, 1), lambda i: (i, 0)),
                  pl.BlockSpec((BN, 1), lambda i: (i, 0))],
        out_specs=[pl.BlockSpec((BN, PF), lambda i: (i, 0)),
                   pl.BlockSpec((BN, Fo), lambda i: (i, 0))],
        out_shape=[jax.ShapeDtypeStruct((N, PF), F32),
                   jax.ShapeDtypeStruct((N, Fo), F32)],
    )(hw, dis_col, sl_col)


def _fin_body(m_ref, so_ref, b_ref, o_ref, *, act):
    Fo = so_ref.shape[1]
    m = m_ref[...]
    s = m[0, :, :Fo] + m[1, :, :Fo] + so_ref[...] + b_ref[...]
    if act == "relu":
        o_ref[...] = jnp.maximum(s, 0.0)
    elif act == "lsm":
        mx = jnp.max(s, axis=1, keepdims=True)
        z = s - mx
        o_ref[...] = z - jnp.log(jnp.sum(jnp.exp(z), axis=1, keepdims=True))
    else:
        o_ref[...] = s


def _fin(m2, so, b_row, act):
    Fo = so.shape[1]
    BN = 2000
    return pl.pallas_call(
        functools.partial(_fin_body, act=act),
        grid=(N // BN,),
        in_specs=[pl.BlockSpec((NC, BN, PF), lambda i: (0, i, 0)),
                  pl.BlockSpec((BN, Fo), lambda i: (i, 0)),
                  pl.BlockSpec((1, Fo), lambda i: (0, 0))],
        out_specs=pl.BlockSpec((BN, Fo), lambda i: (i, 0)),
        out_shape=jax.ShapeDtypeStruct((N, Fo), F32),
    )(m2, so, b_row)


def _mesh():
    return plsc.VectorSubcoreMesh(core_axis_name="c", subcore_axis_name="s")


_SC_PARAMS = pltpu.CompilerParams(needs_layout_passes=False)


def _copy_idx(src, dst):
    for q in range(C // 16):
        dst[pl.ds(q * 16, 16)] = src[pl.ds(q * 16, 16)]


def _k1(F, xn, row, col, zn):
    @functools.partial(
        pl.kernel,
        out_type=[jax.ShapeDtypeStruct((EP,), F32),
                  jax.ShapeDtypeStruct((NC, N), F32),
                  jax.ShapeDtypeStruct((NC, N), F32)],
        mesh=_mesh(),
        compiler_params=_SC_PARAMS,
        scratch_types=[
            pltpu.VMEM((C,), I32), pltpu.VMEM((C,), I32),
            pltpu.VMEM((C,), I32), pltpu.VMEM((C,), I32),
            pltpu.VMEM((C, PF), F32), pltpu.VMEM((C, PF), F32),
            pltpu.VMEM((C, PF), F32), pltpu.VMEM((C, PF), F32),
            pltpu.VMEM((C,), F32), pltpu.VMEM((C,), F32),
            pltpu.VMEM_SHARED((N,), F32),
            pltpu.VMEM_SHARED((N,), F32),
            pltpu.SemaphoreType.DMA, pltpu.SemaphoreType.DMA,
            pltpu.SemaphoreType.DMA,
        ],
    )
    def body(xn_hbm, row_hbm, col_hbm, zn_hbm, simt_hbm, rs_hbm, dg_hbm,
             ridx0, cidx0, ridx1, cidx1,
             rowsA0, rowsB0, rowsA1, rowsB1,
             simv0, indv0,
             rs_sh, dg_sh, semI, semG0, semG1):
        cid = lax.axis_index("c")
        sid = lax.axis_index("s")
        wid = sid * NC + cid

        @pl.when(sid == 0)
        def _():
            pltpu.sync_copy(zn_hbm, rs_sh)
            pltpu.sync_copy(zn_hbm, dg_sh)

        plsc.subcore_barrier()

        def in_base(k):
            return wid * EPW + k * C

        def p_base(k):
            return wid * EPAD + k * C

        def load_idx(k, r, c):
            c1 = pltpu.async_copy(row_hbm.at[pl.ds(in_base(k), C)], r, semI)
            c2 = pltpu.async_copy(col_hbm.at[pl.ds(in_base(k), C)], c, semI)
            c1.wait()
            c2.wait()

        na = 8 if F % 8 == 0 else 4
        lane = jnp.arange(16, dtype=I32)

        def compute(k, rA, rB, sv, iv):
            def group(g, c2):
                e0 = g * 16
                ei = lane + e0
                zero = jnp.zeros((16,), F32)

                def jstep(jv, accs):
                    res = list(accs)
                    j0 = jv * na
                    for t in range(na):
                        jj = (lane + (j0 + t)) % F
                        a = plsc.load_gather(rA, [ei, jj])
                        b = plsc.load_gather(rB, [ei, jj])
                        res[t] = res[t] + a * b
                    return tuple(res)

                accs = lax.fori_loop(0, F // na, jstep,
                                     tuple(zero for _ in range(na)))
                acc = accs[0]
                for t in range(1, na):
                    acc = acc + accs[t]
                valid = k * C + e0 < EPW
                st = jnp.where(valid & (acc >= P0), acc, 0.0)
                sv[pl.ds(e0, 16)] = st
                iv[pl.ds(e0, 16)] = jnp.where(st > 0.0, 1.0, 0.0)
                return c2

            lax.fori_loop(0, G, group, 0)

        def pairc(t, carry):
            a = 2 * t
            b = a + 1
            load_idx(a, ridx0, cidx0)
            cpA0 = pltpu.async_copy(xn_hbm.at[ridx0], rowsA0, semG0)
            cpB0 = pltpu.async_copy(xn_hbm.at[cidx0], rowsB0, semG0)
            load_idx(b, ridx1, cidx1)
            cpA1 = pltpu.async_copy(xn_hbm.at[ridx1], rowsA1, semG1)
            cpB1 = pltpu.async_copy(xn_hbm.at[cidx1], rowsB1, semG1)
            cpA0.wait()
            cpB0.wait()
            compute(a, rowsA0, rowsB0, simv0, indv0)
            pltpu.sync_copy(simv0, simt_hbm.at[pl.ds(p_base(a), C)])
            pltpu.sync_copy(simv0, rs_sh.at[ridx0], add=True)
            pltpu.sync_copy(indv0, dg_sh.at[ridx0], add=True)
            cpA1.wait()
            cpB1.wait()
            compute(b, rowsA1, rowsB1, simv0, indv0)
            pltpu.sync_copy(simv0, simt_hbm.at[pl.ds(p_base(b), C)])
            pltpu.sync_copy(simv0, rs_sh.at[ridx1], add=True)
            pltpu.sync_copy(indv0, dg_sh.at[ridx1], add=True)
            return carry

        lax.fori_loop(0, NCH2, pairc, 0)
        plsc.subcore_barrier()

        @pl.when(sid == 0)
        def _():
            pltpu.sync_copy(rs_sh, rs_hbm.at[cid])
            pltpu.sync_copy(dg_sh, dg_hbm.at[cid])

    return body(xn, row, col, zn)


def _k2(simt, row, col, denom, zn):
    @functools.partial(
        pl.kernel,
        out_type=[jax.ShapeDtypeStruct((EP,), F32),
                  jax.ShapeDtypeStruct((NC, N), F32)],
        mesh=_mesh(),
        compiler_params=_SC_PARAMS,
        scratch_types=[
            pltpu.VMEM((C,), I32),
            pltpu.VMEM((C,), I32),
            pltpu.VMEM((C,), F32),
            pltpu.VMEM((C,), F32),
            pltpu.VMEM((N,), F32),
            pltpu.VMEM_SHARED((N,), F32),
            pltpu.SemaphoreType.DMA,
        ],
    )
    def body(simt_hbm, row_hbm, col_hbm, den_hbm, zn_hbm, ew_hbm, dc_hbm,
             ridx, cidx, sv, ewv, denv, dc_sh, semI):
        cid = lax.axis_index("c")
        sid = lax.axis_index("s")
        wid = sid * NC + cid

        pltpu.sync_copy(den_hbm, denv)

        @pl.when(sid == 0)
        def _():
            pltpu.sync_copy(zn_hbm, dc_sh)

        plsc.subcore_barrier()

        def chunk(i, carry):
            base = wid * EPW + i * C
            basep = wid * EPAD + i * C
            c1 = pltpu.async_copy(row_hbm.at[pl.ds(base, C)], ridx, semI)
            c2 = pltpu.async_copy(col_hbm.at[pl.ds(base, C)], cidx, semI)
            c3 = pltpu.async_copy(simt_hbm.at[pl.ds(basep, C)], sv, semI)
            c1.wait()
            c2.wait()
            c3.wait()

            def group(g, c2_):
                e0 = g * 16
                rv = ridx[pl.ds(e0, 16)]
                s = sv[pl.ds(e0, 16)]
                d = plsc.load_gather(denv, [rv])
                w = s / d
                ewv[pl.ds(e0, 16)] = jnp.where(s > 0.0, jnp.exp(w), 0.0)
                return c2_

            lax.fori_loop(0, G, group, 0)
            pltpu.sync_copy(ewv, ew_hbm.at[pl.ds(basep, C)])
            pltpu.sync_copy(ewv, dc_sh.at[cidx], add=True)
            return carry

        lax.fori_loop(0, NCHP, chunk, 0)
        plsc.subcore_barrier()

        @pl.when(sid == 0)
        def _():
            pltpu.sync_copy(dc_sh, dc_hbm.at[cid])

    return body(simt, row, col, denom, zn)


def _k3(Fo, hws, ew, row, col, dis, znf):
    @functools.partial(
        pl.kernel,
        out_type=jax.ShapeDtypeStruct((NC, N, PF), F32),
        mesh=_mesh(),
        compiler_params=_SC_PARAMS,
        scratch_types=[
            pltpu.VMEM((C,), I32), pltpu.VMEM((C,), I32),
            pltpu.VMEM((C,), I32), pltpu.VMEM((C,), I32),
            pltpu.VMEM((C, PF), F32), pltpu.VMEM((C, PF), F32),
            pltpu.VMEM((C,), F32), pltpu.VMEM((C,), F32),
            pltpu.VMEM((C,), F32),
            pltpu.VMEM((N,), F32),
            pltpu.VMEM_SHARED((N, PF), F32),
            pltpu.SemaphoreType.DMA,
            pltpu.SemaphoreType.DMA, pltpu.SemaphoreType.DMA,
        ],
    )
    def body(hws_hbm, ew_hbm, row_hbm, col_hbm, dis_hbm, znf_hbm, msg_hbm,
             ridx0, cidx0, ridx1, cidx1,
             rows0, rows1,
             ewv0, ewv1, scalev,
             disv, out_sh, semI, semG0, semG1):
        cid = lax.axis_index("c")
        sid = lax.axis_index("s")
        wid = sid * NC + cid

        pltpu.sync_copy(dis_hbm, disv)

        @pl.when(sid == 0)
        def _():
            pltpu.sync_copy(znf_hbm, out_sh)

        plsc.subcore_barrier()

        def in_base(k):
            return wid * EPW + k * C

        def p_base(k):
            return wid * EPAD + k * C

        def load_in(k, r, c, e):
            c1 = pltpu.async_copy(row_hbm.at[pl.ds(in_base(k), C)], r, semI)
            c2 = pltpu.async_copy(col_hbm.at[pl.ds(in_base(k), C)], c, semI)
            c3 = pltpu.async_copy(ew_hbm.at[pl.ds(p_base(k), C)], e, semI)
            c1.wait()
            c2.wait()
            c3.wait()

        nsl = -(-Fo // 16)

        def compute(r_rows, cidx, ewv):
            def group(g, c2_):
                e0 = g * 16
                cv = cidx[pl.ds(e0, 16)]
                sc = ewv[pl.ds(e0, 16)] * plsc.load_gather(disv, [cv])
                scalev[pl.ds(e0, 16)] = sc
                return c2_

            lax.fori_loop(0, G, group, 0)

            def edge(e, c2_):
                sc = plsc.load_gather(scalev, [jnp.full((16,), e, I32)])
                r = r_rows.at[e]
                for k in range(nsl):
                    r[pl.ds(k * 16, 16)] = r[pl.ds(k * 16, 16)] * sc
                return c2_

            lax.fori_loop(0, C, edge, 0)

        def pairc(t, carry):
            a = 2 * t
            b = a + 1
            load_in(a, ridx0, cidx0, ewv0)
            g0 = pltpu.async_copy(hws_hbm.at[ridx0], rows0, semG0)
            load_in(b, ridx1, cidx1, ewv1)
            g1 = pltpu.async_copy(hws_hbm.at[ridx1], rows1, semG1)
            g0.wait()
            compute(rows0, cidx0, ewv0)
            pltpu.sync_copy(rows0, out_sh.at[cidx0], add=True)
            g1.wait()
            compute(rows1, cidx1, ewv1)
            pltpu.sync_copy(rows1, out_sh.at[cidx1], add=True)
            return carry

        lax.fori_loop(0, NCH3, pairc, 0)
        plsc.subcore_barrier()

        @pl.when(sid == 0)
        def _():
            pltpu.sync_copy(out_sh, msg_hbm.at[cid])

    return body(hws, ew, row, col, dis, znf)


def kernel(x, adj_indices, W1, b1, W2, b2, W3, b3):
    row = jnp.pad(adj_indices[0], (0, PADIN))
    col = jnp.pad(adj_indices[1], (0, PADIN))
    zn = jnp.zeros((N,), F32)
    h = x
    for li, (W, b) in enumerate(((W1, b1), (W2, b2), (W3, b3))):
        F, Fo = h.shape[1], W.shape[1]
        xn, hw = _prep(h, W)
        simt, rs2, dg2 = _k1(F, xn, row, col, zn)
        den, slw = _mid(rs2, dg2)
        ew, dc2 = _k2(simt, row, col, den.reshape(N), zn)
        dis = _post1(dc2, slw)
        hws, so = _post2(hw, dis.reshape(N, 1), slw.reshape(N, 1))
        znf = jnp.zeros((N, PF), F32)
        m2 = _k3(Fo, hws, ew, row, col, dis.reshape(N), znf)
        h = _fin(m2, so, b.reshape(1, Fo), "lsm" if li == 2 else "relu")
    return h
